# baseline (speedup 1.0000x reference)
import functools

import jax
import jax.numpy as jnp
from jax import lax
from jax.experimental import pallas as pl
from jax.experimental.pallas import tpu as pltpu
from jax.experimental.pallas import tpu_sc as plsc

_NUM_CORES = 2
_NUM_SUBCORES = 16
_NW = _NUM_CORES * _NUM_SUBCORES
_LANES = 16
_GROUP = 8


@functools.partial(jax.jit, static_argnums=(2, 3, 4))
def _sc_gather(table, idx_p, N, L, D):
    LP = idx_p.shape[1]
    r_per_w = N // _NW
    HALF = r_per_w // 2
    n_half = HALF * L
    CH = 1600
    n_ch = n_half // CH
    n_grp = HALF // _GROUP
    PER_G = _GROUP * L // _LANES
    mesh = plsc.VectorSubcoreMesh(core_axis_name="c", subcore_axis_name="s")

    @functools.partial(
        pl.kernel,
        out_type=jax.ShapeDtypeStruct((N * L, D), jnp.float32),
        mesh=mesh,
        compiler_params=pltpu.CompilerParams(
            use_tc_tiling_on_sc=False, needs_layout_passes=False
        ),
        scratch_types=[
            pltpu.VMEM((HALF, LP), jnp.int32),
            pltpu.VMEM((n_half,), jnp.int32),
            pltpu.VMEM((CH, D), jnp.float32),
            pltpu.SemaphoreType.DMA,
        ],
    )
    def gather_kernel(t_hbm, idx_hbm, out_hbm, slab_v, idxc_v, rows_v, sem):
        wid = lax.axis_index("s") * _NUM_CORES + lax.axis_index("c")
        rbase = wid * r_per_w
        lane = lax.iota(jnp.int32, _LANES)
        row_off = []
        col_off = []
        for t in range(PER_G):
            b0 = (_LANES * t) // L
            r = (_LANES * t) % L
            if r + _LANES <= L:
                row_off.append(lane * 0 + b0)
                col_off.append(lane + r)
            else:
                cross = L - r
                in_first = lane < cross
                row_off.append(
                    jnp.where(in_first, b0, b0 + 1).astype(jnp.int32)
                )
                col_off.append(jnp.where(in_first, lane + r, lane - cross))

        for h in range(2):
            pltpu.sync_copy(idx_hbm.at[pl.ds(rbase + h * HALF, HALF)], slab_v)

            def compact(j, carry):
                r0 = j * _GROUP
                p0 = j * (_GROUP * L)
                for t in range(PER_G):
                    vals = plsc.load_gather(
                        slab_v, [r0 + row_off[t], col_off[t]]
                    )
                    idxc_v[pl.ds(p0 + t * _LANES, _LANES)] = vals
                return carry

            lax.fori_loop(0, n_grp, compact, 0)

            def body(c, carry):
                off = c * CH
                pltpu.async_copy(
                    t_hbm.at[idxc_v.at[pl.ds(off, CH)]], rows_v, sem
                ).wait()
                pltpu.sync_copy(
                    rows_v,
                    out_hbm.at[pl.ds((rbase + h * HALF) * L + off, CH)],
                )
                return carry

            lax.fori_loop(0, n_ch, body, 0)

    return gather_kernel(table, idx_p)


def _adapter_body(g_ref, w_ref, b_ref, o_ref):
    w = w_ref[...]
    bb = b_ref[...]
    for l in range(o_ref.shape[0]):
        x = g_ref[:, l, :]
        y = jnp.dot(x, w, preferred_element_type=jnp.float32) + bb
        o_ref[l, :, :] = y.T


def _adapter(g3, W, b, N, L, D):
    NB = 256
    assert N % NB == 0
    return pl.pallas_call(
        _adapter_body,
        grid=(N // NB,),
        in_specs=[
            pl.BlockSpec((NB, L, D), lambda i: (i, 0, 0)),
            pl.BlockSpec((D, D), lambda i: (0, 0)),
            pl.BlockSpec((1, D), lambda i: (0, 0)),
        ],
        out_specs=pl.BlockSpec((L, D, NB), lambda i: (0, 0, i)),
        out_shape=jax.ShapeDtypeStruct((L, D, N), jnp.float32),
    )(g3, W, b.reshape(1, D))


def kernel(indices, table, W, b):
    V, D = table.shape
    N, L = indices.shape
    idx_p = jnp.pad(indices.astype(jnp.int32), ((0, 0), (0, 128 - L)))
    g = _sc_gather(table, idx_p, N, L, D)
    g3 = g.reshape(N, L, D)
    out_t = _adapter(g3, W, b, N, L, D)
    return jnp.transpose(out_t, (2, 0, 1))
